# Initial kernel scaffold; baseline (speedup 1.0000x reference)
import functools

import jax
import jax.numpy as jnp
from jax import lax
from jax.experimental import pallas as pl
from jax.experimental.pallas import tpu as pltpu
from jax.experimental.pallas import tpu_sc as plsc

N = 10000
E = 320000
D = 128
C = 40
CP = 48

NC = 2
NS = 16
NW = NC * NS
EPW = E // NW
CH = 100
NCH = EPW // CH


def _proj_body(x_ref, wl_ref, y_ref):
    y = jnp.dot(x_ref[...], wl_ref[...], preferred_element_type=jnp.float32)
    col = lax.broadcasted_iota(jnp.int32, y.shape, 1)
    y_ref[...] = y + jnp.where(col == C, 1.0, 0.0)


def _project(x, wl48):
    blk = 2000
    return pl.pallas_call(
        _proj_body,
        grid=(N // blk,),
        in_specs=[
            pl.BlockSpec((blk, D), lambda i: (i, 0)),
            pl.BlockSpec((D, CP), lambda i: (0, 0)),
        ],
        out_specs=pl.BlockSpec((blk, CP), lambda i: (i, 0)),
        out_shape=jax.ShapeDtypeStruct((N, CP), jnp.float32),
    )(x, wl48)


def _sc_aggregate(y48, src, dst, zeros):
    mesh = plsc.VectorSubcoreMesh(core_axis_name="c", subcore_axis_name="s")

    @functools.partial(
        pl.kernel,
        mesh=mesh,
        out_type=[
            jax.ShapeDtypeStruct((N, CP), jnp.float32),
            jax.ShapeDtypeStruct((N, CP), jnp.float32),
        ],
        scratch_types=[
            pltpu.VMEM((NCH, CH), jnp.int32),
            pltpu.VMEM((NCH, CH), jnp.int32),
            pltpu.VMEM((CH, CP), jnp.float32),
            pltpu.VMEM((CH, CP), jnp.float32),
            pltpu.VMEM_SHARED((N, CP), jnp.float32),
            pltpu.SemaphoreType.DMA,
            pltpu.SemaphoreType.DMA,
        ],
    )
    def k(y_hbm, src_hbm, dst_hbm, z_hbm, out0, out1,
          src_v, dst_v, buf0, buf1, agg, sem0, sem1):
        cid = lax.axis_index("c")
        sid = lax.axis_index("s")
        wid = cid * NS + sid
        rows_per = N // NS
        stripe = pl.ds(sid * rows_per, rows_per)
        pltpu.sync_copy(z_hbm.at[stripe], agg.at[stripe])
        pltpu.sync_copy(src_hbm.at[wid], src_v)
        pltpu.sync_copy(dst_hbm.at[wid], dst_v)
        plsc.subcore_barrier()

        pltpu.async_copy(y_hbm.at[src_v.at[0]], buf0, sem0)

        def body(t, carry):
            j0 = 2 * t
            pltpu.async_copy(y_hbm.at[src_v.at[j0 + 1]], buf1, sem1)
            pltpu.make_async_copy(y_hbm.at[src_v.at[j0]], buf0, sem0).wait()
            pltpu.sync_copy(buf0, agg.at[dst_v.at[j0]], add=True)

            @pl.when(j0 + 2 < NCH)
            def _():
                pltpu.async_copy(y_hbm.at[src_v.at[j0 + 2]], buf0, sem0)

            pltpu.make_async_copy(y_hbm.at[src_v.at[j0 + 1]], buf1, sem1).wait()
            pltpu.sync_copy(buf1, agg.at[dst_v.at[j0 + 1]], add=True)
            return carry

        lax.fori_loop(0, NCH // 2, body, 0)
        plsc.subcore_barrier()

        @pl.when(cid == 0)
        def _():
            pltpu.sync_copy(agg.at[stripe], out0.at[stripe])

        @pl.when(cid == 1)
        def _():
            pltpu.sync_copy(agg.at[stripe], out1.at[stripe])

    return k(y48, src, dst, zeros)


def _combine_body(p0_ref, p1_ref, x_ref, wr_ref, b_ref, o_ref):
    agg = p0_ref[...] + p1_ref[...]
    col = lax.broadcasted_iota(jnp.int32, agg.shape, 1)
    cnt = jnp.sum(jnp.where(col == C, agg, 0.0), axis=1, keepdims=True)
    mean = agg / jnp.maximum(cnt, 1.0)
    z = jnp.dot(x_ref[...], wr_ref[...], preferred_element_type=jnp.float32)
    out48 = jnp.maximum(mean + z + b_ref[...], 0.0)
    o_ref[...] = out48[:, :C]


def _combine(p0, p1, x, wr48, b48):
    blk = 2000
    return pl.pallas_call(
        _combine_body,
        grid=(N // blk,),
        in_specs=[
            pl.BlockSpec((blk, CP), lambda i: (i, 0)),
            pl.BlockSpec((blk, CP), lambda i: (i, 0)),
            pl.BlockSpec((blk, D), lambda i: (i, 0)),
            pl.BlockSpec((D, CP), lambda i: (0, 0)),
            pl.BlockSpec((1, CP), lambda i: (0, 0)),
        ],
        out_specs=pl.BlockSpec((blk, C), lambda i: (i, 0)),
        out_shape=jax.ShapeDtypeStruct((N, C), jnp.float32),
    )(p0, p1, x, wr48, b48)


def kernel(x, edge_index, W_l, W_r, b):
    ei = edge_index.astype(jnp.int32)
    src = ei[0].reshape(NW, NCH, CH)
    dst = ei[1].reshape(NW, NCH, CH)
    wl48 = jnp.pad(W_l, ((0, 0), (0, CP - C)))
    wr48 = jnp.pad(W_r, ((0, 0), (0, CP - C)))
    b48 = jnp.pad(b, (0, CP - C)).reshape(1, CP)
    zeros = jnp.zeros((N, CP), jnp.float32)
    y48 = _project(x, wl48)
    p0, p1 = _sc_aggregate(y48, src, dst, zeros)
    return _combine(p0, p1, x, wr48, b48)


# speedup vs baseline: 16.1564x; 16.1564x over previous
import functools

import jax
import jax.numpy as jnp
from jax import lax
from jax.experimental import pallas as pl
from jax.experimental.pallas import tpu as pltpu
from jax.experimental.pallas import tpu_sc as plsc

N = 10000
E = 320000
D = 128
C = 40
CP = 48

NC = 2
NS = 16
NW = NC * NS
EPW = E // NW
CH = 100
NCH = EPW // CH


def _proj_body(x_ref, wl_ref, y_ref):
    y = jnp.dot(x_ref[...], wl_ref[...], preferred_element_type=jnp.float32)
    col = lax.broadcasted_iota(jnp.int32, y.shape, 1)
    y_ref[...] = y + jnp.where(col == C, 1.0, 0.0)


def _project(x, wl48):
    blk = 2000
    return pl.pallas_call(
        _proj_body,
        grid=(N // blk,),
        in_specs=[
            pl.BlockSpec((blk, D), lambda i: (i, 0)),
            pl.BlockSpec((D, CP), lambda i: (0, 0)),
        ],
        out_specs=pl.BlockSpec((blk, CP), lambda i: (i, 0)),
        out_shape=jax.ShapeDtypeStruct((N, CP), jnp.float32),
    )(x, wl48)


def _sc_aggregate(y48, src, dst, zeros):
    mesh = plsc.VectorSubcoreMesh(core_axis_name="c", subcore_axis_name="s")

    @functools.partial(
        pl.kernel,
        mesh=mesh,
        compiler_params=pltpu.CompilerParams(use_tc_tiling_on_sc=False),
        out_type=[
            jax.ShapeDtypeStruct((N, CP), jnp.float32),
            jax.ShapeDtypeStruct((N, CP), jnp.float32),
        ],
        scratch_types=[
            pltpu.VMEM((NCH, CH), jnp.int32),
            pltpu.VMEM((NCH, CH), jnp.int32),
            pltpu.VMEM((CH, CP), jnp.float32),
            pltpu.VMEM((CH, CP), jnp.float32),
            pltpu.VMEM_SHARED((N, CP), jnp.float32),
            pltpu.SemaphoreType.DMA,
            pltpu.SemaphoreType.DMA,
        ],
    )
    def k(y_hbm, src_hbm, dst_hbm, z_hbm, out0, out1,
          src_v, dst_v, buf0, buf1, agg, sem0, sem1):
        cid = lax.axis_index("c")
        sid = lax.axis_index("s")
        wid = cid * NS + sid
        st_lo = pl.ds(pl.multiple_of(sid * 624, 8), 624)
        st_hi = pl.ds(15 * 624, N - 15 * 624)

        @pl.when(sid < 15)
        def _():
            pltpu.sync_copy(z_hbm.at[st_lo], agg.at[st_lo])

        @pl.when(sid == 15)
        def _():
            pltpu.sync_copy(z_hbm.at[st_hi], agg.at[st_hi])

        pltpu.sync_copy(src_hbm.at[wid], src_v)
        pltpu.sync_copy(dst_hbm.at[wid], dst_v)
        plsc.subcore_barrier()

        pltpu.async_copy(y_hbm.at[src_v.at[0]], buf0, sem0)

        def body(t, carry):
            j0 = 2 * t
            pltpu.async_copy(y_hbm.at[src_v.at[j0 + 1]], buf1, sem1)
            pltpu.make_async_copy(y_hbm.at[src_v.at[j0]], buf0, sem0).wait()
            pltpu.sync_copy(buf0, agg.at[dst_v.at[j0]], add=True)

            @pl.when(j0 + 2 < NCH)
            def _():
                pltpu.async_copy(y_hbm.at[src_v.at[j0 + 2]], buf0, sem0)

            pltpu.make_async_copy(y_hbm.at[src_v.at[j0 + 1]], buf1, sem1).wait()
            pltpu.sync_copy(buf1, agg.at[dst_v.at[j0 + 1]], add=True)
            return carry

        lax.fori_loop(0, NCH // 2, body, 0)
        plsc.subcore_barrier()

        @pl.when(jnp.logical_and(cid == 0, sid < 15))
        def _():
            pltpu.sync_copy(agg.at[st_lo], out0.at[st_lo])

        @pl.when(jnp.logical_and(cid == 0, sid == 15))
        def _():
            pltpu.sync_copy(agg.at[st_hi], out0.at[st_hi])

        @pl.when(jnp.logical_and(cid == 1, sid < 15))
        def _():
            pltpu.sync_copy(agg.at[st_lo], out1.at[st_lo])

        @pl.when(jnp.logical_and(cid == 1, sid == 15))
        def _():
            pltpu.sync_copy(agg.at[st_hi], out1.at[st_hi])

    return k(y48, src, dst, zeros)


def _combine_body(p0_ref, p1_ref, x_ref, wr_ref, b_ref, o_ref):
    agg = p0_ref[...] + p1_ref[...]
    col = lax.broadcasted_iota(jnp.int32, agg.shape, 1)
    cnt = jnp.sum(jnp.where(col == C, agg, 0.0), axis=1, keepdims=True)
    mean = agg / jnp.maximum(cnt, 1.0)
    z = jnp.dot(x_ref[...], wr_ref[...], preferred_element_type=jnp.float32)
    out48 = jnp.maximum(mean + z + b_ref[...], 0.0)
    o_ref[...] = out48[:, :C]


def _combine(p0, p1, x, wr48, b48):
    blk = 2000
    return pl.pallas_call(
        _combine_body,
        grid=(N // blk,),
        in_specs=[
            pl.BlockSpec((blk, CP), lambda i: (i, 0)),
            pl.BlockSpec((blk, CP), lambda i: (i, 0)),
            pl.BlockSpec((blk, D), lambda i: (i, 0)),
            pl.BlockSpec((D, CP), lambda i: (0, 0)),
            pl.BlockSpec((1, CP), lambda i: (0, 0)),
        ],
        out_specs=pl.BlockSpec((blk, C), lambda i: (i, 0)),
        out_shape=jax.ShapeDtypeStruct((N, C), jnp.float32),
    )(p0, p1, x, wr48, b48)


def kernel(x, edge_index, W_l, W_r, b):
    ei = edge_index.astype(jnp.int32)
    src = ei[0].reshape(NW, NCH, CH)
    dst = ei[1].reshape(NW, NCH, CH)
    wl48 = jnp.pad(W_l, ((0, 0), (0, CP - C)))
    wr48 = jnp.pad(W_r, ((0, 0), (0, CP - C)))
    b48 = jnp.pad(b, (0, CP - C)).reshape(1, CP)
    zeros = jnp.zeros((N, CP), jnp.float32)
    y48 = _project(x, wl48)
    p0, p1 = _sc_aggregate(y48, src, dst, zeros)
    return _combine(p0, p1, x, wr48, b48)


# speedup vs baseline: 17.4791x; 1.0819x over previous
import functools

import jax
import jax.numpy as jnp
from jax import lax
from jax.experimental import pallas as pl
from jax.experimental.pallas import tpu as pltpu
from jax.experimental.pallas import tpu_sc as plsc

N = 10000
E = 320000
D = 128
C = 40
CP = 48

NC = 2
NS = 16
NW = NC * NS
CH = 128
NCHT = E // CH
CPW = NCHT // NW
NEPI = NCHT - CPW * NW


def _proj_body(x_ref, wl_ref, y_ref):
    y = jnp.dot(x_ref[...], wl_ref[...], preferred_element_type=jnp.float32)
    col = lax.broadcasted_iota(jnp.int32, y.shape, 1)
    y_ref[...] = y + jnp.where(col == C, 1.0, 0.0)


def _project(x, wl48):
    blk = 2000
    return pl.pallas_call(
        _proj_body,
        grid=(N // blk,),
        in_specs=[
            pl.BlockSpec((blk, D), lambda i: (i, 0)),
            pl.BlockSpec((D, CP), lambda i: (0, 0)),
        ],
        out_specs=pl.BlockSpec((blk, CP), lambda i: (i, 0)),
        out_shape=jax.ShapeDtypeStruct((N, CP), jnp.float32),
    )(x, wl48)


def _sc_aggregate(y48, src, dst, zeros):
    mesh = plsc.VectorSubcoreMesh(core_axis_name="c", subcore_axis_name="s")

    @functools.partial(
        pl.kernel,
        mesh=mesh,
        compiler_params=pltpu.CompilerParams(use_tc_tiling_on_sc=False),
        out_type=[
            jax.ShapeDtypeStruct((N, CP), jnp.float32),
            jax.ShapeDtypeStruct((N, CP), jnp.float32),
        ],
        scratch_types=[
            pltpu.VMEM((CPW + 1, CH), jnp.int32),
            pltpu.VMEM((CPW + 1, CH), jnp.int32),
            pltpu.VMEM((CH, CP), jnp.float32),
            pltpu.VMEM((CH, CP), jnp.float32),
            pltpu.VMEM_SHARED((N, CP), jnp.float32),
            pltpu.SemaphoreType.DMA,
            pltpu.SemaphoreType.DMA,
        ],
    )
    def k(y_hbm, src_hbm, dst_hbm, z_hbm, out0, out1,
          src_v, dst_v, buf0, buf1, agg, sem0, sem1):
        cid = lax.axis_index("c")
        sid = lax.axis_index("s")
        wid = cid * NS + sid
        st_lo = pl.ds(pl.multiple_of(sid * 624, 8), 624)
        st_hi = pl.ds(15 * 624, N - 15 * 624)

        @pl.when(sid < 15)
        def _():
            pltpu.sync_copy(z_hbm.at[st_lo], agg.at[st_lo])

        @pl.when(sid == 15)
        def _():
            pltpu.sync_copy(z_hbm.at[st_hi], agg.at[st_hi])

        pltpu.sync_copy(src_hbm.at[pl.ds(wid * CPW, CPW)], src_v.at[pl.ds(0, CPW)])
        pltpu.sync_copy(dst_hbm.at[pl.ds(wid * CPW, CPW)], dst_v.at[pl.ds(0, CPW)])

        @pl.when(wid < NEPI)
        def _():
            pltpu.sync_copy(src_hbm.at[pl.ds(NW * CPW + wid, 1)],
                            src_v.at[pl.ds(CPW, 1)])
            pltpu.sync_copy(dst_hbm.at[pl.ds(NW * CPW + wid, 1)],
                            dst_v.at[pl.ds(CPW, 1)])

        plsc.subcore_barrier()

        pltpu.async_copy(y_hbm.at[src_v.at[0]], buf0, sem0)

        def body(t, carry):
            j0 = 2 * t
            pltpu.async_copy(y_hbm.at[src_v.at[j0 + 1]], buf1, sem1)
            pltpu.make_async_copy(y_hbm.at[src_v.at[j0]], buf0, sem0).wait()
            pltpu.sync_copy(buf0, agg.at[dst_v.at[j0]], add=True)

            @pl.when(j0 + 2 < CPW)
            def _():
                pltpu.async_copy(y_hbm.at[src_v.at[j0 + 2]], buf0, sem0)

            pltpu.make_async_copy(y_hbm.at[src_v.at[j0 + 1]], buf1, sem1).wait()
            pltpu.sync_copy(buf1, agg.at[dst_v.at[j0 + 1]], add=True)
            return carry

        lax.fori_loop(0, CPW // 2, body, 0)

        @pl.when(wid < NEPI)
        def _():
            pltpu.async_copy(y_hbm.at[src_v.at[CPW]], buf0, sem0).wait()
            pltpu.sync_copy(buf0, agg.at[dst_v.at[CPW]], add=True)

        plsc.subcore_barrier()

        @pl.when(jnp.logical_and(cid == 0, sid < 15))
        def _():
            pltpu.sync_copy(agg.at[st_lo], out0.at[st_lo])

        @pl.when(jnp.logical_and(cid == 0, sid == 15))
        def _():
            pltpu.sync_copy(agg.at[st_hi], out0.at[st_hi])

        @pl.when(jnp.logical_and(cid == 1, sid < 15))
        def _():
            pltpu.sync_copy(agg.at[st_lo], out1.at[st_lo])

        @pl.when(jnp.logical_and(cid == 1, sid == 15))
        def _():
            pltpu.sync_copy(agg.at[st_hi], out1.at[st_hi])

    return k(y48, src, dst, zeros)


def _combine_body(p0_ref, p1_ref, x_ref, wr_ref, b_ref, o_ref):
    agg = p0_ref[...] + p1_ref[...]
    col = lax.broadcasted_iota(jnp.int32, agg.shape, 1)
    cnt = jnp.sum(jnp.where(col == C, agg, 0.0), axis=1, keepdims=True)
    mean = agg / jnp.maximum(cnt, 1.0)
    z = jnp.dot(x_ref[...], wr_ref[...], preferred_element_type=jnp.float32)
    out48 = jnp.maximum(mean + z + b_ref[...], 0.0)
    o_ref[...] = out48[:, :C]


def _combine(p0, p1, x, wr48, b48):
    blk = 2000
    return pl.pallas_call(
        _combine_body,
        grid=(N // blk,),
        in_specs=[
            pl.BlockSpec((blk, CP), lambda i: (i, 0)),
            pl.BlockSpec((blk, CP), lambda i: (i, 0)),
            pl.BlockSpec((blk, D), lambda i: (i, 0)),
            pl.BlockSpec((D, CP), lambda i: (0, 0)),
            pl.BlockSpec((1, CP), lambda i: (0, 0)),
        ],
        out_specs=pl.BlockSpec((blk, C), lambda i: (i, 0)),
        out_shape=jax.ShapeDtypeStruct((N, C), jnp.float32),
    )(p0, p1, x, wr48, b48)


def kernel(x, edge_index, W_l, W_r, b):
    ei = edge_index.astype(jnp.int32)
    src = ei[0].reshape(NCHT, CH)
    dst = ei[1].reshape(NCHT, CH)
    wl48 = jnp.pad(W_l, ((0, 0), (0, CP - C)))
    wr48 = jnp.pad(W_r, ((0, 0), (0, CP - C)))
    b48 = jnp.pad(b, (0, CP - C)).reshape(1, CP)
    zeros = jnp.zeros((N, CP), jnp.float32)
    y48 = _project(x, wl48)
    p0, p1 = _sc_aggregate(y48, src, dst, zeros)
    return _combine(p0, p1, x, wr48, b48)


# speedup vs baseline: 21.4509x; 1.2272x over previous
import functools

import jax
import jax.numpy as jnp
from jax import lax
from jax.experimental import pallas as pl
from jax.experimental.pallas import tpu as pltpu
from jax.experimental.pallas import tpu_sc as plsc

N = 10000
E = 320000
D = 128
C = 40
CP = 48
W = 128

NC = 2
NS = 16
NW = NC * NS
EPW = E // NW
BC = 384
NBC = EPW // BC
TAIL = EPW - NBC * BC


def _proj_body(x_ref, wl_ref, y_ref):
    y = jnp.dot(x_ref[...], wl_ref[...], preferred_element_type=jnp.float32)
    col = lax.broadcasted_iota(jnp.int32, y.shape, 1)
    y_ref[...] = y + jnp.where(col == C, 1.0, 0.0)


def _project(x, wl48):
    blk = 2000
    return pl.pallas_call(
        _proj_body,
        grid=(N // blk,),
        in_specs=[
            pl.BlockSpec((blk, D), lambda i: (i, 0)),
            pl.BlockSpec((D, CP), lambda i: (0, 0)),
        ],
        out_specs=pl.BlockSpec((blk, CP), lambda i: (i, 0)),
        out_shape=jax.ShapeDtypeStruct((N, CP), jnp.float32),
    )(x, wl48)


def _sc_aggregate(y, src, dst, zeros):
    mesh = plsc.VectorSubcoreMesh(core_axis_name="c", subcore_axis_name="s")

    @functools.partial(
        pl.kernel,
        mesh=mesh,
        compiler_params=pltpu.CompilerParams(use_tc_tiling_on_sc=False),
        out_type=[
            jax.ShapeDtypeStruct((N, W), jnp.float32),
            jax.ShapeDtypeStruct((N, W), jnp.float32),
        ],
        scratch_types=[
            pltpu.VMEM((EPW,), jnp.int32),
            pltpu.VMEM((EPW,), jnp.int32),
            pltpu.VMEM((BC, CP), jnp.float32),
            pltpu.VMEM((BC, CP), jnp.float32),
            pltpu.VMEM_SHARED((N, CP), jnp.float32),
            pltpu.SemaphoreType.DMA,
            pltpu.SemaphoreType.DMA,
        ],
    )
    def k(y_hbm, src_hbm, dst_hbm, z_hbm, out0, out1,
          src_v, dst_v, buf0, buf1, agg, sem0, sem1):
        cid = lax.axis_index("c")
        sid = lax.axis_index("s")
        wid = cid * NS + sid
        st_lo = pl.ds(pl.multiple_of(sid * 624, 8), 624)
        st_hi = pl.ds(15 * 624, N - 15 * 624)

        @pl.when(sid < 15)
        def _():
            pltpu.sync_copy(z_hbm.at[st_lo], agg.at[st_lo])

        @pl.when(sid == 15)
        def _():
            pltpu.sync_copy(z_hbm.at[st_hi], agg.at[st_hi])

        pltpu.sync_copy(src_hbm.at[pl.ds(wid * EPW, EPW)], src_v)
        pltpu.sync_copy(dst_hbm.at[pl.ds(wid * EPW, EPW)], dst_v)
        plsc.subcore_barrier()

        pltpu.async_copy(y_hbm.at[src_v.at[pl.ds(0, BC)]], buf0, sem0)

        def body(t, carry):
            r0 = 2 * BC * t
            pltpu.async_copy(y_hbm.at[src_v.at[pl.ds(r0 + BC, BC)]], buf1, sem1)
            pltpu.make_async_copy(
                y_hbm.at[src_v.at[pl.ds(r0, BC)]], buf0, sem0).wait()
            pltpu.sync_copy(buf0, agg.at[dst_v.at[pl.ds(r0, BC)]], add=True)

            @pl.when(r0 + 2 * BC < NBC * BC)
            def _():
                pltpu.async_copy(
                    y_hbm.at[src_v.at[pl.ds(r0 + 2 * BC, BC)]], buf0, sem0)

            pltpu.make_async_copy(
                y_hbm.at[src_v.at[pl.ds(r0 + BC, BC)]], buf1, sem1).wait()
            pltpu.sync_copy(buf1, agg.at[dst_v.at[pl.ds(r0 + BC, BC)]], add=True)
            return carry

        lax.fori_loop(0, NBC // 2, body, 0)

        bslice = buf0.at[pl.ds(0, TAIL)]
        pltpu.async_copy(
            y_hbm.at[src_v.at[pl.ds(NBC * BC, TAIL)]], bslice, sem0).wait()
        pltpu.sync_copy(bslice, agg.at[dst_v.at[pl.ds(NBC * BC, TAIL)]], add=True)

        plsc.subcore_barrier()

        @pl.when(jnp.logical_and(cid == 0, sid < 15))
        def _():
            pltpu.sync_copy(agg.at[st_lo], out0.at[st_lo, pl.ds(0, CP)])

        @pl.when(jnp.logical_and(cid == 0, sid == 15))
        def _():
            pltpu.sync_copy(agg.at[st_hi], out0.at[st_hi, pl.ds(0, CP)])

        @pl.when(jnp.logical_and(cid == 1, sid < 15))
        def _():
            pltpu.sync_copy(agg.at[st_lo], out1.at[st_lo, pl.ds(0, CP)])

        @pl.when(jnp.logical_and(cid == 1, sid == 15))
        def _():
            pltpu.sync_copy(agg.at[st_hi], out1.at[st_hi, pl.ds(0, CP)])

    return k(y, src, dst, zeros)


def _combine_body(p0_ref, p1_ref, x_ref, wr_ref, b_ref, o_ref):
    agg = p0_ref[...] + p1_ref[...]
    col = lax.broadcasted_iota(jnp.int32, agg.shape, 1)
    cnt = jnp.sum(jnp.where(col == C, agg, 0.0), axis=1, keepdims=True)
    mean = agg / jnp.maximum(cnt, 1.0)
    z = jnp.dot(x_ref[...], wr_ref[...], preferred_element_type=jnp.float32)
    o_ref[...] = jnp.maximum(mean[:, :C] + z + b_ref[...], 0.0)


def _combine(p0, p1, x, wr, b40):
    blk = 2000
    return pl.pallas_call(
        _combine_body,
        grid=(N // blk,),
        in_specs=[
            pl.BlockSpec((blk, W), lambda i: (i, 0)),
            pl.BlockSpec((blk, W), lambda i: (i, 0)),
            pl.BlockSpec((blk, D), lambda i: (i, 0)),
            pl.BlockSpec((D, C), lambda i: (0, 0)),
            pl.BlockSpec((1, C), lambda i: (0, 0)),
        ],
        out_specs=pl.BlockSpec((blk, C), lambda i: (i, 0)),
        out_shape=jax.ShapeDtypeStruct((N, C), jnp.float32),
    )(p0, p1, x, wr, b40)


def kernel(x, edge_index, W_l, W_r, b):
    ei = edge_index.astype(jnp.int32)
    src = ei[0]
    dst = ei[1]
    wl48 = jnp.pad(W_l, ((0, 0), (0, CP - C)))
    b40 = b.reshape(1, C)
    zeros = jnp.zeros((N, CP), jnp.float32)
    y = _project(x, wl48)
    p0, p1 = _sc_aggregate(y, src, dst, zeros)
    return _combine(p0, p1, x, W_r, b40)


# speedup vs baseline: 21.4998x; 1.0023x over previous
import functools

import jax
import jax.numpy as jnp
from jax import lax
from jax.experimental import pallas as pl
from jax.experimental.pallas import tpu as pltpu
from jax.experimental.pallas import tpu_sc as plsc

N = 10000
E = 320000
D = 128
C = 40
CP = 48
W = 128

NC = 2
NS = 16
NW = NC * NS
EPW = E // NW
BC = 624
NBC = EPW // BC
TAIL = EPW - NBC * BC


def _proj_body(x_ref, wl_ref, y_ref):
    y = jnp.dot(x_ref[...], wl_ref[...], preferred_element_type=jnp.float32)
    col = lax.broadcasted_iota(jnp.int32, y.shape, 1)
    y_ref[...] = y + jnp.where(col == C, 1.0, 0.0)


def _project(x, wl48):
    blk = 2000
    return pl.pallas_call(
        _proj_body,
        grid=(N // blk,),
        in_specs=[
            pl.BlockSpec((blk, D), lambda i: (i, 0)),
            pl.BlockSpec((D, CP), lambda i: (0, 0)),
        ],
        out_specs=pl.BlockSpec((blk, CP), lambda i: (i, 0)),
        out_shape=jax.ShapeDtypeStruct((N, CP), jnp.float32),
    )(x, wl48)


def _sc_aggregate(y, src, dst, zeros):
    mesh = plsc.VectorSubcoreMesh(core_axis_name="c", subcore_axis_name="s")

    @functools.partial(
        pl.kernel,
        mesh=mesh,
        compiler_params=pltpu.CompilerParams(use_tc_tiling_on_sc=False),
        out_type=[
            jax.ShapeDtypeStruct((N, W), jnp.float32),
            jax.ShapeDtypeStruct((N, W), jnp.float32),
        ],
        scratch_types=[
            pltpu.VMEM((EPW,), jnp.int32),
            pltpu.VMEM((EPW,), jnp.int32),
            pltpu.VMEM((BC, CP), jnp.float32),
            pltpu.VMEM((BC, CP), jnp.float32),
            pltpu.VMEM_SHARED((N, CP), jnp.float32),
            pltpu.SemaphoreType.DMA,
            pltpu.SemaphoreType.DMA,
        ],
    )
    def k(y_hbm, src_hbm, dst_hbm, z_hbm, out0, out1,
          src_v, dst_v, buf0, buf1, agg, sem0, sem1):
        cid = lax.axis_index("c")
        sid = lax.axis_index("s")
        wid = cid * NS + sid
        st_lo = pl.ds(pl.multiple_of(sid * 624, 8), 624)
        st_hi = pl.ds(15 * 624, N - 15 * 624)

        @pl.when(sid < 15)
        def _():
            pltpu.sync_copy(z_hbm.at[pl.ds(0, 624)], agg.at[st_lo])

        @pl.when(sid == 15)
        def _():
            pltpu.sync_copy(z_hbm, agg.at[st_hi])

        pltpu.sync_copy(src_hbm.at[pl.ds(wid * EPW, EPW)], src_v)
        pltpu.sync_copy(dst_hbm.at[pl.ds(wid * EPW, EPW)], dst_v)
        plsc.subcore_barrier()

        pltpu.async_copy(y_hbm.at[src_v.at[pl.ds(0, BC)]], buf0, sem0)

        def body(t, carry):
            r0 = 2 * BC * t
            pltpu.async_copy(y_hbm.at[src_v.at[pl.ds(r0 + BC, BC)]], buf1, sem1)
            pltpu.make_async_copy(
                y_hbm.at[src_v.at[pl.ds(r0, BC)]], buf0, sem0).wait()
            pltpu.sync_copy(buf0, agg.at[dst_v.at[pl.ds(r0, BC)]], add=True)

            @pl.when(r0 + 2 * BC < NBC * BC)
            def _():
                pltpu.async_copy(
                    y_hbm.at[src_v.at[pl.ds(r0 + 2 * BC, BC)]], buf0, sem0)

            pltpu.make_async_copy(
                y_hbm.at[src_v.at[pl.ds(r0 + BC, BC)]], buf1, sem1).wait()
            pltpu.sync_copy(buf1, agg.at[dst_v.at[pl.ds(r0 + BC, BC)]], add=True)
            return carry

        lax.fori_loop(0, NBC // 2, body, 0)

        bslice = buf0.at[pl.ds(0, TAIL)]
        pltpu.async_copy(
            y_hbm.at[src_v.at[pl.ds(NBC * BC, TAIL)]], bslice, sem0).wait()
        pltpu.sync_copy(bslice, agg.at[dst_v.at[pl.ds(NBC * BC, TAIL)]], add=True)

        plsc.subcore_barrier()

        @pl.when(jnp.logical_and(cid == 0, sid < 15))
        def _():
            pltpu.sync_copy(agg.at[st_lo], out0.at[st_lo, pl.ds(0, CP)])

        @pl.when(jnp.logical_and(cid == 0, sid == 15))
        def _():
            pltpu.sync_copy(agg.at[st_hi], out0.at[st_hi, pl.ds(0, CP)])

        @pl.when(jnp.logical_and(cid == 1, sid < 15))
        def _():
            pltpu.sync_copy(agg.at[st_lo], out1.at[st_lo, pl.ds(0, CP)])

        @pl.when(jnp.logical_and(cid == 1, sid == 15))
        def _():
            pltpu.sync_copy(agg.at[st_hi], out1.at[st_hi, pl.ds(0, CP)])

    return k(y, src, dst, zeros)


def _combine_body(p0_ref, p1_ref, x_ref, wr_ref, b_ref, o_ref):
    agg = p0_ref[...] + p1_ref[...]
    col = lax.broadcasted_iota(jnp.int32, agg.shape, 1)
    cnt = jnp.sum(jnp.where(col == C, agg, 0.0), axis=1, keepdims=True)
    mean = agg / jnp.maximum(cnt, 1.0)
    z = jnp.dot(x_ref[...], wr_ref[...], preferred_element_type=jnp.float32)
    o_ref[...] = jnp.maximum(mean[:, :C] + z + b_ref[...], 0.0)


def _combine(p0, p1, x, wr, b40):
    blk = 2000
    return pl.pallas_call(
        _combine_body,
        grid=(N // blk,),
        in_specs=[
            pl.BlockSpec((blk, W), lambda i: (i, 0)),
            pl.BlockSpec((blk, W), lambda i: (i, 0)),
            pl.BlockSpec((blk, D), lambda i: (i, 0)),
            pl.BlockSpec((D, C), lambda i: (0, 0)),
            pl.BlockSpec((1, C), lambda i: (0, 0)),
        ],
        out_specs=pl.BlockSpec((blk, C), lambda i: (i, 0)),
        out_shape=jax.ShapeDtypeStruct((N, C), jnp.float32),
    )(p0, p1, x, wr, b40)


def kernel(x, edge_index, W_l, W_r, b):
    ei = edge_index.astype(jnp.int32)
    src = ei[0]
    dst = ei[1]
    wl48 = jnp.pad(W_l, ((0, 0), (0, CP - C)))
    b40 = b.reshape(1, C)
    zeros = jnp.zeros((640, CP), jnp.float32)
    y = _project(x, wl48)
    p0, p1 = _sc_aggregate(y, src, dst, zeros)
    return _combine(p0, p1, x, W_r, b40)
